# Initial kernel scaffold; baseline (speedup 1.0000x reference)
import jax
import jax.numpy as jnp
from jax.experimental import pallas as pl


def kernel(logits, box_masks):
    raise NotImplementedError("write your pallas kernel here")


# speedup vs baseline: 1.0222x; 1.0222x over previous
import jax
import jax.numpy as jnp
from jax import lax
from jax.experimental import pallas as pl

MINIMUM = 0.1
MAXIMUM = 0.9
SLICES_WIDTH = 4


def _pen(v):
    return jnp.where(v >= 0, v * v, 0.0)


def _plane_kernel(lg_ref, bm_ref, out_ref):
    w = SLICES_WIDTH
    lg = lg_ref[0]
    Wd, Hd = lg.shape
    nW, nH = Wd // w, Hd // w

    r_ids = lax.broadcasted_iota(jnp.int32, (nW, Wd), 1) // w
    slab_ids = lax.broadcasted_iota(jnp.int32, (nW, Wd), 0)
    A_rows = (r_ids == slab_ids).astype(jnp.float32)
    c_ids = lax.broadcasted_iota(jnp.int32, (Hd, nH), 0) // w
    slab_ids_c = lax.broadcasted_iota(jnp.int32, (Hd, nH), 1)
    A_cols = (c_ids == slab_ids_c).astype(jnp.float32)

    total = 0.0
    union_sum = jnp.zeros_like(lg)
    for n in range(bm_ref.shape[1]):
        m = bm_ref[0, n].astype(jnp.float32)
        ml = lg * m
        union_sum = union_sum + m

        row_ml = jnp.sum(ml, axis=1, keepdims=True)
        col_ml = jnp.sum(ml, axis=0, keepdims=True)
        row_m = jnp.sum(m, axis=1, keepdims=True)
        col_m = jnp.sum(m, axis=0, keepdims=True)

        sw = jnp.dot(A_rows, row_ml,
                     preferred_element_type=jnp.float32)
        mw_s = jnp.dot(A_rows, row_m,
                       preferred_element_type=jnp.float32)
        sh = jnp.dot(col_ml, A_cols,
                     preferred_element_type=jnp.float32)
        mh_s = jnp.dot(col_m, A_cols,
                       preferred_element_type=jnp.float32)

        mw = (mw_s > 0).astype(jnp.float32)
        mh = (mh_s > 0).astype(jnp.float32)

        actual = jnp.sum(sw)
        box = jnp.sum(row_m)

        size_err = _pen(actual - MAXIMUM * box) + _pen(MINIMUM * box - actual)
        tight = jnp.sum(_pen(w - sw) * mw) + jnp.sum(_pen(w - sh) * mh)
        total = total + size_err + tight

    union = (union_sum > 0).astype(jnp.float32)
    outside = lg * (1.0 - union)
    total = total + jnp.sum(_pen(outside))
    out_ref[0, 0, :] = jnp.full((out_ref.shape[-1],), total, jnp.float32)


def kernel(logits, box_masks):
    lg = logits[:, 1:]
    bm = box_masks[:, 1:]
    B, Cf, Wd, Hd = lg.shape
    N = bm.shape[2]
    P = B * Cf
    lg = lg.reshape(P, Wd, Hd)
    bm = bm.reshape(P, N, Wd, Hd).astype(jnp.int8)

    partials = pl.pallas_call(
        _plane_kernel,
        grid=(P,),
        in_specs=[
            pl.BlockSpec((1, Wd, Hd), lambda i: (i, 0, 0)),
            pl.BlockSpec((1, N, Wd, Hd), lambda i: (i, 0, 0, 0)),
        ],
        out_specs=pl.BlockSpec((1, 1, 128), lambda i: (i, 0, 0)),
        out_shape=jax.ShapeDtypeStruct((P, 1, 128), jnp.float32),
    )(lg, bm)

    im_prod = Cf * Wd * Hd
    return jnp.sum(partials[:, 0, 0]) / im_prod
